# baseline (speedup 1.0000x reference)
import functools

import jax
import jax.numpy as jnp
from jax import lax
from jax.experimental import pallas as pl
from jax.experimental.pallas import tpu as pltpu
from jax.experimental.pallas import tpu_sc as plsc

_BATCH = 16384
_EMBED = 16
_HOUR_VOCAB = 24
_PHONE_VOCAB = 100000
_NSUP = _PHONE_VOCAB // 8
_NSUP_PAD = _NSUP + 4
_NC = 2
_NS = 16
_NW = _NC * _NS
_B_PER_W = _BATCH // _NW
_G = 16
_STRIP = 256


@jax.jit
def _context_embedding_sc(hour_idx, phone_idx, hour_table, pt_pad):
    mesh = plsc.VectorSubcoreMesh(core_axis_name="c", subcore_axis_name="s")

    @functools.partial(
        pl.kernel,
        mesh=mesh,
        out_type=jax.ShapeDtypeStruct((_BATCH * 2 * _EMBED // 128, 128),
                                      jnp.float32),
        scratch_types=[
            pltpu.VMEM((_HOUR_VOCAB, _EMBED), jnp.float32),
            pltpu.VMEM((_B_PER_W,), jnp.int32),
            pltpu.VMEM((_B_PER_W,), jnp.int32),
            pltpu.VMEM((_B_PER_W,), jnp.int32),
            pltpu.VMEM((_B_PER_W, 128), jnp.float32),
            pltpu.VMEM((_STRIP * 2 * _EMBED // 128, 128), jnp.float32),
            pltpu.SemaphoreType.DMA,
        ],
    )
    def k(hi_hbm, pi_hbm, ht_hbm, pt_hbm, out_hbm,
          ht_v, hi_v, pi_v, psup_v, prows_v, cat_v, sem):
        wid = lax.axis_index("s") * _NC + lax.axis_index("c")
        base = wid * _B_PER_W
        pltpu.sync_copy(hi_hbm.at[pl.ds(base, _B_PER_W)], hi_v)
        pltpu.sync_copy(pi_hbm.at[pl.ds(base, _B_PER_W)], pi_v)
        pltpu.sync_copy(ht_hbm, ht_v)

        @pl.loop(0, _B_PER_W // _G)
        def _(g):
            psup_v.at[pl.ds(g * _G, _G)][...] = (
                pi_v[pl.ds(g * _G, _G)] >> 3)

        gp = pltpu.async_copy(pt_hbm.at[psup_v], prows_v, sem)
        gp.wait()

        for s in range(_B_PER_W // _STRIP):
            @pl.loop(0, _STRIP // _G)
            def _(g):
                e0 = s * _STRIP + g * _G
                hvec = hi_v[pl.ds(e0, _G)]
                pvec = (pi_v[pl.ds(e0, _G)] & 7) * _EMBED
                for j in range(_G):
                    i = g * _G + j
                    row, col = i // 4, (i % 4) * 32
                    cat_v.at[row, pl.ds(col, _EMBED)][...] = (
                        ht_v.at[hvec[j], pl.ds(0, _EMBED)][...])
                    cat_v.at[row, pl.ds(col + _EMBED, _EMBED)][...] = (
                        prows_v.at[s * _STRIP + i, pl.ds(pvec[j], _EMBED)][...])

            pltpu.sync_copy(
                cat_v,
                out_hbm.at[pl.ds(
                    pl.multiple_of((base + s * _STRIP) // 4, 8),
                    _STRIP // 4)])

    return k(hour_idx, phone_idx, hour_table, pt_pad)


def kernel(hour_idx, phone_idx, hour_table, phone_table):
    pt_pad = jnp.concatenate(
        [phone_table.reshape(_NSUP, 128),
         jnp.zeros((_NSUP_PAD - _NSUP, 128), jnp.float32)])
    out_wide = _context_embedding_sc(
        hour_idx.astype(jnp.int32),
        phone_idx.astype(jnp.int32),
        hour_table,
        pt_pad,
    )
    return out_wide.reshape(_BATCH, 2 * _EMBED)


# speedup vs baseline: 1.0612x; 1.0612x over previous
import functools

import jax
import jax.numpy as jnp
from jax import lax
from jax.experimental import pallas as pl
from jax.experimental.pallas import tpu as pltpu
from jax.experimental.pallas import tpu_sc as plsc

_BATCH = 16384
_EMBED = 16
_HOUR_VOCAB = 24
_PHONE_VOCAB = 100000
_NSUP = _PHONE_VOCAB // 8
_NSUP_PAD = _NSUP + 4
_NC = 2
_NS = 16
_NW = _NC * _NS
_B_PER_W = _BATCH // _NW
_G = 16
_STRIP = 256


@jax.jit
def _context_embedding_sc(hour_idx, phone_idx, hour_table, pt_wide):
    mesh = plsc.VectorSubcoreMesh(core_axis_name="c", subcore_axis_name="s")

    @functools.partial(
        pl.kernel,
        mesh=mesh,
        out_type=jax.ShapeDtypeStruct((_BATCH * 2 * _EMBED // 128, 128),
                                      jnp.float32),
        scratch_types=[
            pltpu.VMEM((_HOUR_VOCAB, _EMBED), jnp.float32),
            pltpu.VMEM((_B_PER_W,), jnp.int32),
            pltpu.VMEM((_B_PER_W,), jnp.int32),
            pltpu.VMEM((_B_PER_W,), jnp.int32),
            pltpu.VMEM((_B_PER_W, 128), jnp.float32),
            pltpu.VMEM((_STRIP * 2 * _EMBED // 128, 128), jnp.float32),
            pltpu.SemaphoreType.DMA,
        ],
    )
    def k(hi_hbm, pi_hbm, ht_hbm, pt_hbm, out_hbm,
          ht_v, hi_v, pi_v, psup_v, prows_v, cat_v, sem):
        wid = lax.axis_index("s") * _NC + lax.axis_index("c")
        base = wid * _B_PER_W
        pltpu.sync_copy(hi_hbm.at[pl.ds(base, _B_PER_W)], hi_v)
        pltpu.sync_copy(pi_hbm.at[pl.ds(base, _B_PER_W)], pi_v)
        pltpu.sync_copy(ht_hbm, ht_v)

        @pl.loop(0, _B_PER_W // _G)
        def _(g):
            psup_v.at[pl.ds(g * _G, _G)][...] = (
                pi_v[pl.ds(g * _G, _G)] >> 3)

        gp = pltpu.async_copy(pt_hbm.at[psup_v], prows_v, sem)
        gp.wait()

        for s in range(_B_PER_W // _STRIP):
            @pl.loop(0, _STRIP // _G)
            def _(g):
                e0 = s * _STRIP + g * _G
                hvec = hi_v[pl.ds(e0, _G)]
                pvec = (pi_v[pl.ds(e0, _G)] & 7) * _EMBED
                for j in range(_G):
                    i = g * _G + j
                    row, col = i // 4, (i % 4) * 32
                    cat_v.at[row, pl.ds(col, _EMBED)][...] = (
                        ht_v.at[hvec[j], pl.ds(0, _EMBED)][...])
                    cat_v.at[row, pl.ds(col + _EMBED, _EMBED)][...] = (
                        prows_v.at[s * _STRIP + i, pl.ds(pvec[j], _EMBED)][...])

            pltpu.sync_copy(
                cat_v,
                out_hbm.at[pl.ds(
                    pl.multiple_of((base + s * _STRIP) // 4, 8),
                    _STRIP // 4)])

    return k(hour_idx, phone_idx, hour_table, pt_wide)


def kernel(hour_idx, phone_idx, hour_table, phone_table):
    out_wide = _context_embedding_sc(
        hour_idx.astype(jnp.int32),
        phone_idx.astype(jnp.int32),
        hour_table,
        phone_table.reshape(_NSUP, 128),
    )
    return out_wide.reshape(_BATCH, 2 * _EMBED)
